# baseline (speedup 1.0000x reference)
import jax
import jax.numpy as jnp
from jax import lax
from jax.experimental import pallas as pl
from jax.experimental.pallas import tpu as pltpu
from jax.experimental.pallas import tpu_sc as plsc

H = 1024
L = 16
NC = 2
NS = 16
NW = NC * NS
B = 4 * 2048
RPW = B // NW
G = 16
NCH = RPW // G
HC = H // L
EPS = 1e-5


_GATHER_DNUMS = lax.GatherDimensionNumbers(
    offset_dims=(), collapsed_slice_dims=(0,), start_index_map=(0,))


def _shuffle(v, perm):
    return lax.gather(v, perm[:, None], _GATHER_DNUMS, (1,),
                      mode=lax.GatherScatterMode.PROMISE_IN_BOUNDS)


def _rsqrt(x):
    i = lax.bitcast_convert_type(x, jnp.int32)
    i = jnp.int32(0x5F3759DF) - lax.shift_right_logical(i, 1)
    y = lax.bitcast_convert_type(i, jnp.float32)
    for _ in range(3):
        y = y * (1.5 - 0.5 * x * y * y)
    return y


def _body(tok_hbm, table_hbm, w_hbm, b_hbm, out_hbm,
          idx_v, rows_v, w_v, b_v, gsem):
    wid = lax.axis_index("s") * NC + lax.axis_index("c")
    base = wid * RPW
    pltpu.sync_copy(tok_hbm.at[pl.ds(base, RPW)], idx_v)
    pltpu.sync_copy(w_hbm, w_v)
    pltpu.sync_copy(b_hbm, b_v)

    def chunk_body(g, carry):
        idx_reg = idx_v[pl.ds(g * G, G)]
        pltpu.async_copy(table_hbm.at[idx_reg], rows_v, gsem).wait()

        def row_body(r, carry2):
            def red(c, acc):
                v = rows_v[r, pl.ds(c * L, L)]
                return acc[0] + v, acc[1] + v * v
            zero = jnp.zeros((L,), jnp.float32)
            acc, acc2 = lax.fori_loop(0, HC, red, (zero, zero))
            lanes = lax.iota(jnp.int32, L)
            for k in (8, 4, 2, 1):
                perm = lanes ^ k
                acc = acc + _shuffle(acc, perm)
                acc2 = acc2 + _shuffle(acc2, perm)
            mean_v = acc * (1.0 / H)
            var_v = acc2 * (1.0 / H) - mean_v * mean_v
            rstd_v = _rsqrt(var_v + EPS)

            def norm(c, carry3):
                sl = pl.ds(c * L, L)
                v = rows_v[r, sl]
                rows_v[r, sl] = (v - mean_v) * rstd_v * w_v[sl] + b_v[sl]
                return carry3
            return lax.fori_loop(0, HC, norm, carry2)

        lax.fori_loop(0, G, row_body, 0)
        pltpu.sync_copy(rows_v, out_hbm.at[pl.ds(base + g * G, G)])
        return carry

    lax.fori_loop(0, NCH, chunk_body, 0)


@jax.jit
def kernel(input_token, table, ln_weight, ln_bias):
    bsz, seq = input_token.shape
    tok = input_token.reshape(-1).astype(jnp.int32)
    mesh = plsc.VectorSubcoreMesh(core_axis_name="c", subcore_axis_name="s")
    k = pl.kernel(
        _body,
        out_type=jax.ShapeDtypeStruct((B, H), jnp.float32),
        mesh=mesh,
        scratch_types=[
            pltpu.VMEM((RPW,), jnp.int32),
            pltpu.VMEM((G, H), jnp.float32),
            pltpu.VMEM((H,), jnp.float32),
            pltpu.VMEM((H,), jnp.float32),
            pltpu.SemaphoreType.DMA,
        ],
    )
    out = k(tok, table, ln_weight, ln_bias)
    return out.reshape(bsz, seq, H)


# speedup vs baseline: 3.2925x; 3.2925x over previous
import jax
import jax.numpy as jnp
from jax import lax
from jax.experimental import pallas as pl
from jax.experimental.pallas import tpu as pltpu
from jax.experimental.pallas import tpu_sc as plsc

H = 1024
L = 16
NC = 2
NS = 16
NW = NC * NS
B = 4 * 2048
RPW = B // NW
G = 16
NCH = RPW // G
HC = H // L
RG = 8
EPS = 1e-5

_GATHER_DNUMS = lax.GatherDimensionNumbers(
    offset_dims=(), collapsed_slice_dims=(0,), start_index_map=(0,))


def _shuffle(v, perm):
    return lax.gather(v, perm[:, None], _GATHER_DNUMS, (1,),
                      mode=lax.GatherScatterMode.PROMISE_IN_BOUNDS)


def _rsqrt(x):
    i = lax.bitcast_convert_type(x, jnp.int32)
    i = jnp.int32(0x5F3759DF) - lax.shift_right_logical(i, 1)
    y = lax.bitcast_convert_type(i, jnp.float32)
    for _ in range(3):
        y = y * (1.5 - 0.5 * x * y * y)
    return y


def _body(tok_hbm, table_hbm, w_hbm, b_hbm, out_hbm,
          idx_v, rows_v, w_v, b_v, gsem0, gsem1, osem0, osem1):
    wid = lax.axis_index("s") * NC + lax.axis_index("c")
    base = wid * RPW
    pltpu.sync_copy(tok_hbm.at[pl.ds(base, RPW)], idx_v)
    pltpu.sync_copy(w_hbm, w_v)
    pltpu.sync_copy(b_hbm, b_v)

    bufs = (rows_v.at[0], rows_v.at[1])
    gsems = (gsem0, gsem1)
    osems = (osem0, osem1)

    def start_gather(g, p):
        pltpu.async_copy(table_hbm.at[idx_v.at[pl.ds(g * G, G)]],
                         bufs[p], gsems[p])

    def wait_gather(p):
        pltpu.make_async_copy(table_hbm.at[idx_v.at[pl.ds(0, G)]],
                              bufs[p], gsems[p]).wait()

    def start_out(g, p):
        pltpu.async_copy(bufs[p], out_hbm.at[pl.ds(base + g * G, G)],
                         osems[p])

    def wait_out(p):
        pltpu.make_async_copy(bufs[p], out_hbm.at[pl.ds(base, G)],
                              osems[p]).wait()

    lanes = lax.iota(jnp.int32, L)
    zero = jnp.zeros((L,), jnp.float32)

    def compute(rows):
        for gr in range(G // RG):
            r0 = gr * RG

            def red(c, carry):
                sl = pl.ds(c * L, L)
                accs = list(carry[:RG])
                acc2s = list(carry[RG:])
                for i in range(RG):
                    v = rows[r0 + i, sl]
                    accs[i] = accs[i] + v
                    acc2s[i] = acc2s[i] + v * v
                return tuple(accs) + tuple(acc2s)

            carry = lax.fori_loop(0, HC, red, (zero,) * (2 * RG))
            means = []
            rstds = []
            for i in range(RG):
                a, a2 = carry[i], carry[RG + i]
                for k in (8, 4, 2, 1):
                    perm = lanes ^ k
                    a = a + _shuffle(a, perm)
                    a2 = a2 + _shuffle(a2, perm)
                m = a * (1.0 / H)
                var = a2 * (1.0 / H) - m * m
                means.append(m)
                rstds.append(_rsqrt(var + EPS))

            def norm(c, carry3):
                sl = pl.ds(c * L, L)
                wv = w_v[sl]
                bv = b_v[sl]
                for i in range(RG):
                    v = rows[r0 + i, sl]
                    rows[r0 + i, sl] = (v - means[i]) * rstds[i] * wv + bv
                return carry3

            lax.fori_loop(0, HC, norm, 0)

    start_gather(0, 0)

    def pair(h, carry):
        g0 = 2 * h
        wait_gather(0)

        @pl.when(h > 0)
        def _():
            wait_out(1)

        start_gather(g0 + 1, 1)
        compute(bufs[0])
        start_out(g0, 0)

        wait_gather(1)

        @pl.when(h < NCH // 2 - 1)
        def _():
            wait_out(0)
            start_gather(g0 + 2, 0)

        compute(bufs[1])
        start_out(g0 + 1, 1)
        return carry

    lax.fori_loop(0, NCH // 2, pair, 0)
    wait_out(0)
    wait_out(1)


@jax.jit
def kernel(input_token, table, ln_weight, ln_bias):
    bsz, seq = input_token.shape
    tok = input_token.reshape(-1).astype(jnp.int32)
    mesh = plsc.VectorSubcoreMesh(core_axis_name="c", subcore_axis_name="s")
    k = pl.kernel(
        _body,
        out_type=jax.ShapeDtypeStruct((B, H), jnp.float32),
        mesh=mesh,
        scratch_types=[
            pltpu.VMEM((RPW,), jnp.int32),
            pltpu.VMEM((2, G, H), jnp.float32),
            pltpu.VMEM((H,), jnp.float32),
            pltpu.VMEM((H,), jnp.float32),
            pltpu.SemaphoreType.DMA,
            pltpu.SemaphoreType.DMA,
            pltpu.SemaphoreType.DMA,
            pltpu.SemaphoreType.DMA,
        ],
    )
    out = k(tok, table, ln_weight, ln_bias)
    return out.reshape(bsz, seq, H)
